# baseline (speedup 1.0000x reference)
import functools

import jax
import jax.numpy as jnp
from jax import lax
from jax.experimental import pallas as pl
from jax.experimental.pallas import tpu as pltpu
from jax.experimental.pallas import tpu_sc as plsc

B = 4096
H = 64
NC = 2
NS = 16
NW = NC * NS
C = 256

N_STATIC = B * 8
N_HIST = B * 200
N_FUT = B * 50
T_FUT = 50
T_PAD = 56
FB = 4

PER_W = (N_STATIC // NW, N_STATIC // NW, N_HIST // NW, N_HIST // NW,
         N_FUT // NW)
IDX_OFF = (0,
           PER_W[0],
           PER_W[0] + PER_W[1],
           PER_W[0] + PER_W[1] + PER_W[2],
           PER_W[0] + PER_W[1] + PER_W[2] + PER_W[3])
IDX_TOTAL = sum(PER_W) + 16
B_PER_W = B // NW


def _body(sc_idx, sca_idx, hc_idx, hca_idx, fu_idx,
          w_sc, w_sca, w_hc, w_hca, w_fu,
          out_s, out_h, out_f,
          idx_v, idx_fu, rows_v, sg0, sg1, sw0, sw1, si):
    w = lax.axis_index("s") * NC + lax.axis_index("c")
    sems_g = (sg0, sg1)
    sems_w = (sw0, sw1)
    idx_streams = (sc_idx, sca_idx, hc_idx, hca_idx, fu_idx)

    for s, idx_hbm in enumerate(idx_streams):
        pltpu.async_copy(idx_hbm.at[pl.ds(w * PER_W[s], PER_W[s])],
                         idx_v.at[pl.ds(IDX_OFF[s], PER_W[s])], si)
    for s, idx_hbm in enumerate(idx_streams):
        pltpu.make_async_copy(idx_hbm.at[pl.ds(0, PER_W[s])],
                              idx_v.at[pl.ds(IDX_OFF[s], PER_W[s])],
                              si).wait()

    lanes = lax.iota(jnp.int32, 16)

    def fu_stage(b, _):
        for j in range(4):
            pos = IDX_OFF[4] + b * T_FUT + j * 16 + lanes
            vals = plsc.load_gather(idx_v, [pos])
            if j == 3:
                vals = jnp.where(lanes < T_FUT - 48, vals, 0)
            idx_fu[b, pl.ds(j * 16, 16)] = vals
        return 0

    lax.fori_loop(0, B_PER_W, fu_stage, 0)

    def pipeline(m, start_chunk, wait_chunk, start_write, wait_write):
        start_chunk(0, 0)
        start_chunk(1, 1)
        wait_chunk(0)
        start_write(0, 0)

        def pair(k, _):
            i0 = 2 * k
            wait_write(0, i0 - 2)
            start_chunk(0, i0)
            wait_chunk(1)
            start_write(1, i0 - 1)

            wait_write(1, i0 - 1)
            start_chunk(1, i0 + 1)
            wait_chunk(0)
            start_write(0, i0)
            return 0

        lax.fori_loop(1, m // 2, pair, 0)
        wait_chunk(1)
        start_write(1, m - 1)
        wait_write(0, m - 2)
        wait_write(1, m - 1)

    def run_stream(s, table, dst_fn):
        per_w = PER_W[s]
        off = IDX_OFF[s]
        base0 = w * per_w

        def start_chunk(buf, i):
            for j in range(C // 128):
                pltpu.async_copy(
                    table.at[idx_v.at[pl.ds(off + i * C + j * 128, 128)]],
                    rows_v.at[buf, pl.ds(j * 128, 128)], sems_g[buf])

        def wait_chunk(buf):
            pltpu.make_async_copy(table.at[pl.ds(0, C)],
                                  rows_v.at[buf], sems_g[buf]).wait()

        def start_write(buf, i):
            pltpu.async_copy(rows_v.at[buf], dst_fn(base0 + i * C, C),
                             sems_w[buf])

        def wait_write(buf, i):
            pltpu.make_async_copy(rows_v.at[buf], dst_fn(base0 + i * C, C),
                                  sems_w[buf]).wait()

        pipeline(per_w // C, start_chunk, wait_chunk, start_write, wait_write)

    def run_future():
        brow0 = w * B_PER_W
        nrows = FB * T_PAD

        def start_chunk(buf, i):
            for b in range(FB):
                pltpu.async_copy(
                    w_fu.at[idx_fu.at[i * FB + b, pl.ds(0, T_PAD)]],
                    rows_v.at[buf, pl.ds(b * T_PAD, T_PAD)], sems_g[buf])

        def wait_chunk(buf):
            pltpu.make_async_copy(w_fu.at[pl.ds(0, nrows)],
                                  rows_v.at[buf, pl.ds(0, nrows)],
                                  sems_g[buf]).wait()

        def start_write(buf, i):
            for b in range(FB):
                pltpu.async_copy(rows_v.at[buf, pl.ds(b * T_PAD, T_PAD)],
                                 out_f.at[brow0 + i * FB + b], sems_w[buf])

        def wait_write(buf, i):
            pltpu.make_async_copy(rows_v.at[buf, pl.ds(0, nrows)],
                                  out_f.at[brow0 + i * FB], sems_w[buf]).wait()

        pipeline(B_PER_W // FB, start_chunk, wait_chunk, start_write,
                 wait_write)

    def interleave(out, parity):
        return lambda base, n: out.at[pl.ds(base, n), parity]

    run_stream(0, w_sc, interleave(out_s, 0))
    run_stream(1, w_sca, interleave(out_s, 1))
    run_stream(2, w_hc, interleave(out_h, 0))
    run_stream(3, w_hca, interleave(out_h, 1))
    run_future()


@jax.jit
def _embed(sc_idx, sca_idx, hc_idx, hca_idx, fu_idx,
           w_sc, w_sca, w_hc, w_hca, w_fu):
    mesh = plsc.VectorSubcoreMesh(core_axis_name="c", subcore_axis_name="s",
                                  num_cores=NC, num_subcores=NS)
    return pl.kernel(
        _body,
        out_type=[
            jax.ShapeDtypeStruct((N_STATIC, 2, H), jnp.float32),
            jax.ShapeDtypeStruct((N_HIST, 2, H), jnp.float32),
            jax.ShapeDtypeStruct((B, T_PAD, H), jnp.float32),
        ],
        mesh=mesh,
        compiler_params=pltpu.CompilerParams(use_tc_tiling_on_sc=False,
                                             needs_layout_passes=False),
        scratch_types=[
            pltpu.VMEM((IDX_TOTAL,), jnp.int32),
            pltpu.VMEM((B_PER_W, 64), jnp.int32),
            pltpu.VMEM((2, C, H), jnp.float32),
            pltpu.SemaphoreType.DMA,
            pltpu.SemaphoreType.DMA,
            pltpu.SemaphoreType.DMA,
            pltpu.SemaphoreType.DMA,
            pltpu.SemaphoreType.DMA,
        ],
    )(sc_idx, sca_idx, hc_idx, hca_idx, fu_idx,
      w_sc, w_sca, w_hc, w_hca, w_fu)


def kernel(static_cont_input, static_cat_input, history_cont_input,
           history_cat_input, future_input, W_static_cont, W_static_cat,
           W_history_cont, W_history_cat, W_future):
    def prep(idx):
        return idx.astype(jnp.int32).reshape(-1)

    out_s, out_h, out_f = _embed(
        prep(static_cont_input), prep(static_cat_input),
        prep(history_cont_input), prep(history_cat_input),
        prep(future_input),
        W_static_cont, W_static_cat, W_history_cont, W_history_cat, W_future)
    return (out_s.reshape(B, 8, 2 * H),
            out_h.reshape(B, 200, 2 * H),
            out_f[:, :T_FUT, :])


# speedup vs baseline: 1.0048x; 1.0048x over previous
import functools

import jax
import jax.numpy as jnp
from jax import lax
from jax.experimental import pallas as pl
from jax.experimental.pallas import tpu as pltpu
from jax.experimental.pallas import tpu_sc as plsc

B = 4096
H = 64
V = 100000
NC = 2
NS = 16
NW = NC * NS
C = 512

N_STATIC = B * 8
N_HIST = B * 200
N_FUT = B * 50

PER_W = (N_STATIC // NW, N_STATIC // NW, N_HIST // NW, N_HIST // NW,
         N_FUT // NW)
IDX_OFF = (0,
           PER_W[0],
           PER_W[0] + PER_W[1],
           PER_W[0] + PER_W[1] + PER_W[2],
           PER_W[0] + PER_W[1] + PER_W[2] + PER_W[3])
IDX_TOTAL = sum(PER_W)

NB_FULL = (V // 128)
V_TAIL = V - NB_FULL * 128
NB_UNIF = NB_FULL // NW * NW
NB_PW = NB_UNIF // NW


def _relayout_body(wt0, wt1, wt2, wt3, wt4, ax0, ax1, ax2, ax3, ax4,
                   p0, p1, p2, p3, p4,
                   vbuf, tbuf, sin, sout):
    w = lax.axis_index("s") * NC + lax.axis_index("c")
    lanes = lax.iota(jnp.int32, 16)
    tables = ((wt0, ax0, p0), (wt1, ax1, p1), (wt2, ax2, p2),
              (wt3, ax3, p3), (wt4, ax4, p4))

    def start_in(wt, buf, vb):
        pltpu.async_copy(wt.at[:, pl.ds(vb * 128, 128)], vbuf.at[buf], sin)

    def wait_in(wt, buf):
        pltpu.make_async_copy(wt.at[:, pl.ds(0, 128)], vbuf.at[buf],
                              sin).wait()

    def transpose(buf):
        def row(pr, _):
            for j in range(8):
                f0 = (16 * j) % 64
                v_local = 2 * pr + (1 if j >= 4 else 0)
                vals = plsc.load_gather(
                    vbuf.at[buf], [f0 + lanes, lanes * 0 + v_local])
                tbuf[buf, pr, pl.ds(16 * j, 16)] = vals
            return 0

        lax.fori_loop(0, 64, row, 0)

    def start_out(pk, buf, vb):
        pltpu.async_copy(tbuf.at[buf], pk.at[pl.ds(vb * 64, 64)], sout)

    def wait_out(pk, buf):
        pltpu.make_async_copy(tbuf.at[buf], pk.at[pl.ds(0, 64)],
                              sout).wait()

    for t, (wt, ax, pk) in enumerate(tables):
        b0 = w * NB_PW

        start_in(wt, 0, b0)
        start_in(wt, 1, b0 + 1)
        wait_in(wt, 0)
        transpose(0)
        start_out(pk, 0, b0)

        def pair(k, _):
            i0 = 2 * k
            wait_out(pk, 0)
            start_in(wt, 0, b0 + i0)
            wait_in(wt, 1)
            transpose(1)
            start_out(pk, 1, b0 + i0 - 1)

            wait_out(pk, 1)
            start_in(wt, 1, b0 + i0 + 1)
            wait_in(wt, 0)
            transpose(0)
            start_out(pk, 0, b0 + i0)
            return 0

        lax.fori_loop(1, NB_PW // 2, pair, 0)

        wait_in(wt, 1)
        transpose(1)
        start_out(pk, 1, b0 + NB_PW - 1)
        wait_out(pk, 0)

        @pl.when(w < NB_FULL - NB_UNIF)
        def _():
            vb = NB_UNIF + w
            start_in(wt, 0, vb)
            wait_in(wt, 0)
            transpose(0)
            start_out(pk, 0, vb)
            wait_out(pk, 0)

        @pl.when(w == 16 + t)
        def _():
            pltpu.sync_copy(ax, vbuf.at[0, pl.ds(0, 16)])
            pltpu.sync_copy(vbuf.at[0, pl.ds(0, 16)],
                            pk.at[pl.ds(NB_FULL * 64, 16)])

        wait_out(pk, 1)


@jax.jit
def _relayout(wt0, wt1, wt2, wt3, wt4, ax0, ax1, ax2, ax3, ax4):
    mesh = plsc.VectorSubcoreMesh(core_axis_name="c", subcore_axis_name="s",
                                  num_cores=NC, num_subcores=NS)
    pk_t = jax.ShapeDtypeStruct((V // 2, 128), jnp.float32)
    return pl.kernel(
        _relayout_body,
        out_type=[pk_t] * 5,
        mesh=mesh,
        compiler_params=pltpu.CompilerParams(use_tc_tiling_on_sc=True,
                                             needs_layout_passes=False),
        scratch_types=[
            pltpu.VMEM((2, 64, 128), jnp.float32),
            pltpu.VMEM((2, 64, 128), jnp.float32),
            pltpu.SemaphoreType.DMA,
            pltpu.SemaphoreType.DMA,
        ],
    )(wt0, wt1, wt2, wt3, wt4, ax0, ax1, ax2, ax3, ax4)


def _body(sc_idx, sca_idx, hc_idx, hca_idx, fu_idx,
          w_sc, w_sca, w_hc, w_hca, w_fu,
          out_s, out_h, out_f,
          idx_v, rows_v, sg0, sg1, sw0, sw1, si):
    w = lax.axis_index("s") * NC + lax.axis_index("c")
    sems_g = (sg0, sg1)
    sems_w = (sw0, sw1)
    idx_streams = (sc_idx, sca_idx, hc_idx, hca_idx, fu_idx)

    for s, idx_hbm in enumerate(idx_streams):
        pltpu.async_copy(idx_hbm.at[pl.ds(w * PER_W[s], PER_W[s])],
                         idx_v.at[pl.ds(IDX_OFF[s], PER_W[s])], si)
    for s, idx_hbm in enumerate(idx_streams):
        pltpu.make_async_copy(idx_hbm.at[pl.ds(0, PER_W[s])],
                              idx_v.at[pl.ds(IDX_OFF[s], PER_W[s])],
                              si).wait()

    def start_chunk(table, off, buf, lbase, n):
        for j in range(n // 128):
            pltpu.async_copy(
                table.at[idx_v.at[pl.ds(off + lbase + j * 128, 128)]],
                rows_v.at[buf, pl.ds(j * 128, 128)], sems_g[buf])

    def wait_chunk(table, buf, n):
        pltpu.make_async_copy(table.at[pl.ds(0, n)],
                              rows_v.at[buf, pl.ds(0, n)],
                              sems_g[buf]).wait()

    def start_write(dst_fn, buf, base, n):
        pltpu.async_copy(rows_v.at[buf, pl.ds(0, n)], dst_fn(base, n),
                         sems_w[buf])

    def wait_write(dst_fn, buf, base, n):
        pltpu.make_async_copy(rows_v.at[buf, pl.ds(0, n)], dst_fn(base, n),
                              sems_w[buf]).wait()

    def run_stream(s, table, dst_fn):
        per_w = PER_W[s]
        off = IDX_OFF[s]
        base0 = w * per_w
        m = per_w // C
        tail = per_w - m * C

        def bofs(i):
            return base0 + i * C

        start_chunk(table, off, 0, 0, C)
        start_chunk(table, off, 1, C, C)
        wait_chunk(table, 0, C)
        start_write(dst_fn, 0, bofs(0), C)

        def pair(k, _):
            i0 = 2 * k
            wait_write(dst_fn, 0, bofs(i0 - 2), C)
            start_chunk(table, off, 0, i0 * C, C)
            wait_chunk(table, 1, C)
            start_write(dst_fn, 1, bofs(i0 - 1), C)

            wait_write(dst_fn, 1, bofs(i0 - 1), C)
            start_chunk(table, off, 1, (i0 + 1) * C, C)
            wait_chunk(table, 0, C)
            start_write(dst_fn, 0, bofs(i0), C)
            return 0

        lax.fori_loop(1, m // 2, pair, 0)

        wait_chunk(table, 1, C)
        start_write(dst_fn, 1, bofs(m - 1), C)
        if tail:
            wait_write(dst_fn, 0, bofs(m - 2), C)
            start_chunk(table, off, 0, m * C, tail)
            wait_chunk(table, 0, tail)
            start_write(dst_fn, 0, bofs(m), tail)
            wait_write(dst_fn, 0, bofs(m), tail)
        else:
            wait_write(dst_fn, 0, bofs(m - 2), C)
        wait_write(dst_fn, 1, bofs(m - 1), C)

    def interleave(out, parity):
        return lambda base, n: out.at[pl.ds(base, n), parity]

    def linear(out):
        return lambda base, n: out.at[pl.ds(base, n)]

    run_stream(0, w_sc, interleave(out_s, 0))
    run_stream(1, w_sca, interleave(out_s, 1))
    run_stream(2, w_hc, interleave(out_h, 0))
    run_stream(3, w_hca, interleave(out_h, 1))
    run_stream(4, w_fu, linear(out_f))


@jax.jit
def _embed(sc_idx, sca_idx, hc_idx, hca_idx, fu_idx,
           w_sc, w_sca, w_hc, w_hca, w_fu):
    mesh = plsc.VectorSubcoreMesh(core_axis_name="c", subcore_axis_name="s",
                                  num_cores=NC, num_subcores=NS)
    return pl.kernel(
        _body,
        out_type=[
            jax.ShapeDtypeStruct((N_STATIC, 2, H), jnp.float32),
            jax.ShapeDtypeStruct((N_HIST, 2, H), jnp.float32),
            jax.ShapeDtypeStruct((N_FUT, H), jnp.float32),
        ],
        mesh=mesh,
        compiler_params=pltpu.CompilerParams(use_tc_tiling_on_sc=False),
        scratch_types=[
            pltpu.VMEM((IDX_TOTAL,), jnp.int32),
            pltpu.VMEM((2, C, H), jnp.float32),
            pltpu.SemaphoreType.DMA,
            pltpu.SemaphoreType.DMA,
            pltpu.SemaphoreType.DMA,
            pltpu.SemaphoreType.DMA,
            pltpu.SemaphoreType.DMA,
        ],
    )(sc_idx, sca_idx, hc_idx, hca_idx, fu_idx,
      w_sc, w_sca, w_hc, w_hca, w_fu)


def kernel(static_cont_input, static_cat_input, history_cont_input,
           history_cat_input, future_input, W_static_cont, W_static_cat,
           W_history_cont, W_history_cat, W_future):
    def prep(idx):
        return idx.astype(jnp.int32).reshape(-1)

    tables = (W_static_cont, W_static_cat, W_history_cont, W_history_cat,
              W_future)
    auxes = tuple(t[NB_FULL * 128:, :].reshape(16, 128) for t in tables)
    packed = _relayout(*(t.T for t in tables), *auxes)
    lin = tuple(p.reshape(V, H) for p in packed)

    out_s, out_h, out_f = _embed(
        prep(static_cont_input), prep(static_cat_input),
        prep(history_cont_input), prep(history_cat_input),
        prep(future_input), *lin)
    return (out_s.reshape(B, 8, 2 * H),
            out_h.reshape(B, 200, 2 * H),
            out_f.reshape(B, 50, H))


# speedup vs baseline: 1.7220x; 1.7138x over previous
import functools

import jax
import jax.numpy as jnp
from jax import lax
from jax.experimental import pallas as pl
from jax.experimental.pallas import tpu as pltpu
from jax.experimental.pallas import tpu_sc as plsc

B = 4096
H = 64
NC = 2
NS = 16
NW = NC * NS
C = 512

N_STATIC = B * 8
N_HIST = B * 200
N_FUT = B * 50

PER_W = (N_STATIC // NW, N_STATIC // NW, N_HIST // NW, N_HIST // NW,
         N_FUT // NW)
IDX_OFF = (0,
           PER_W[0],
           PER_W[0] + PER_W[1],
           PER_W[0] + PER_W[1] + PER_W[2],
           PER_W[0] + PER_W[1] + PER_W[2] + PER_W[3])
IDX_TOTAL = sum(PER_W)


def _body(sc_idx, sca_idx, hc_idx, hca_idx, fu_idx,
          w_sc, w_sca, w_hc, w_hca, w_fu,
          out_s, out_h, out_f,
          idx_v, rows_v, sg0, sg1, sw0, sw1, si):
    w = lax.axis_index("s") * NC + lax.axis_index("c")
    sems_g = (sg0, sg1)
    sems_w = (sw0, sw1)
    idx_streams = (sc_idx, sca_idx, hc_idx, hca_idx, fu_idx)

    for s, idx_hbm in enumerate(idx_streams):
        pltpu.async_copy(idx_hbm.at[pl.ds(w * PER_W[s], PER_W[s])],
                         idx_v.at[pl.ds(IDX_OFF[s], PER_W[s])], si)
    for s, idx_hbm in enumerate(idx_streams):
        pltpu.make_async_copy(idx_hbm.at[pl.ds(0, PER_W[s])],
                              idx_v.at[pl.ds(IDX_OFF[s], PER_W[s])],
                              si).wait()

    def start_chunk(table, off, buf, lbase, n):
        for j in range(n // 128):
            pltpu.async_copy(
                table.at[idx_v.at[pl.ds(off + lbase + j * 128, 128)]],
                rows_v.at[buf, pl.ds(j * 128, 128)], sems_g[buf])

    def wait_chunk(table, buf, n):
        pltpu.make_async_copy(table.at[pl.ds(0, n)],
                              rows_v.at[buf, pl.ds(0, n)],
                              sems_g[buf]).wait()

    def start_write(dst_fn, buf, base, n):
        pltpu.async_copy(rows_v.at[buf, pl.ds(0, n)], dst_fn(base, n),
                         sems_w[buf])

    def wait_write(dst_fn, buf, base, n):
        pltpu.make_async_copy(rows_v.at[buf, pl.ds(0, n)], dst_fn(base, n),
                              sems_w[buf]).wait()

    def run_stream(s, table, dst_fn):
        per_w = PER_W[s]
        off = IDX_OFF[s]
        base0 = w * per_w
        m = per_w // C
        tail = per_w - m * C

        def bofs(i):
            return base0 + i * C

        start_chunk(table, off, 0, 0, C)
        start_chunk(table, off, 1, C, C)
        wait_chunk(table, 0, C)
        start_write(dst_fn, 0, bofs(0), C)

        def pair(k, _):
            i0 = 2 * k
            wait_write(dst_fn, 0, bofs(i0 - 2), C)
            start_chunk(table, off, 0, i0 * C, C)
            wait_chunk(table, 1, C)
            start_write(dst_fn, 1, bofs(i0 - 1), C)

            wait_write(dst_fn, 1, bofs(i0 - 1), C)
            start_chunk(table, off, 1, (i0 + 1) * C, C)
            wait_chunk(table, 0, C)
            start_write(dst_fn, 0, bofs(i0), C)
            return 0

        lax.fori_loop(1, m // 2, pair, 0)

        wait_chunk(table, 1, C)
        start_write(dst_fn, 1, bofs(m - 1), C)
        if tail:
            wait_write(dst_fn, 0, bofs(m - 2), C)
            start_chunk(table, off, 0, m * C, tail)
            wait_chunk(table, 0, tail)
            start_write(dst_fn, 0, bofs(m), tail)
            wait_write(dst_fn, 0, bofs(m), tail)
        else:
            wait_write(dst_fn, 0, bofs(m - 2), C)
        wait_write(dst_fn, 1, bofs(m - 1), C)

    def interleave(out, parity):
        return lambda base, n: out.at[pl.ds(base, n), parity]

    def linear(out):
        return lambda base, n: out.at[pl.ds(base, n)]

    run_stream(0, w_sc, interleave(out_s, 0))
    run_stream(1, w_sca, interleave(out_s, 1))
    run_stream(2, w_hc, interleave(out_h, 0))
    run_stream(3, w_hca, interleave(out_h, 1))
    run_stream(4, w_fu, linear(out_f))


@jax.jit
def _embed(sc_idx, sca_idx, hc_idx, hca_idx, fu_idx,
           w_sc, w_sca, w_hc, w_hca, w_fu):
    mesh = plsc.VectorSubcoreMesh(core_axis_name="c", subcore_axis_name="s",
                                  num_cores=NC, num_subcores=NS)
    return pl.kernel(
        _body,
        out_type=[
            jax.ShapeDtypeStruct((N_STATIC, 2, H), jnp.float32),
            jax.ShapeDtypeStruct((N_HIST, 2, H), jnp.float32),
            jax.ShapeDtypeStruct((N_FUT, H), jnp.float32),
        ],
        mesh=mesh,
        compiler_params=pltpu.CompilerParams(use_tc_tiling_on_sc=False),
        scratch_types=[
            pltpu.VMEM((IDX_TOTAL,), jnp.int32),
            pltpu.VMEM((2, C, H), jnp.float32),
            pltpu.SemaphoreType.DMA,
            pltpu.SemaphoreType.DMA,
            pltpu.SemaphoreType.DMA,
            pltpu.SemaphoreType.DMA,
            pltpu.SemaphoreType.DMA,
        ],
    )(sc_idx, sca_idx, hc_idx, hca_idx, fu_idx,
      w_sc, w_sca, w_hc, w_hca, w_fu)


def kernel(static_cont_input, static_cat_input, history_cont_input,
           history_cat_input, future_input, W_static_cont, W_static_cat,
           W_history_cont, W_history_cat, W_future):
    def prep(idx):
        return idx.astype(jnp.int32).reshape(-1)

    def pack(w):
        return lax.optimization_barrier(w.reshape(-1, 2 * H)).reshape(-1, H)

    out_s, out_h, out_f = _embed(
        prep(static_cont_input), prep(static_cat_input),
        prep(history_cont_input), prep(history_cat_input),
        prep(future_input),
        pack(W_static_cont), pack(W_static_cat), pack(W_history_cont),
        pack(W_history_cat), pack(W_future))
    return (out_s.reshape(B, 8, 2 * H),
            out_h.reshape(B, 200, 2 * H),
            out_f.reshape(B, 50, H))


# speedup vs baseline: 1.8308x; 1.0632x over previous
import functools

import jax
import jax.numpy as jnp
from jax import lax
from jax.experimental import pallas as pl
from jax.experimental.pallas import tpu as pltpu
from jax.experimental.pallas import tpu_sc as plsc

B = 4096
H = 64
NC = 2
NS = 16
NW = NC * NS
C = 512

N_STATIC = B * 8
N_HIST = B * 200
N_FUT = B * 50

_MESH = dict(core_axis_name="c", subcore_axis_name="s",
             num_cores=NC, num_subcores=NS)


def _make_pipeline(idx_v, rows_v, sems_g, sems_w, w):
    def run_stream(table, off, base0, per_w, dst_fn):
        m = per_w // C
        tail = per_w - m * C

        def start_chunk(buf, lbase, n):
            for j in range(n // 128):
                pltpu.async_copy(
                    table.at[idx_v.at[pl.ds(off + lbase + j * 128, 128)]],
                    rows_v.at[buf, pl.ds(j * 128, 128)], sems_g[buf])

        def wait_chunk(buf, n):
            pltpu.make_async_copy(table.at[pl.ds(0, n)],
                                  rows_v.at[buf, pl.ds(0, n)],
                                  sems_g[buf]).wait()

        def start_write(buf, base, n):
            pltpu.async_copy(rows_v.at[buf, pl.ds(0, n)], dst_fn(base, n),
                             sems_w[buf])

        def wait_write(buf, base, n):
            pltpu.make_async_copy(rows_v.at[buf, pl.ds(0, n)],
                                  dst_fn(base, n), sems_w[buf]).wait()

        def bofs(i):
            return base0 + i * C

        start_chunk(0, 0, C)
        start_chunk(1, C, C)
        wait_chunk(0, C)
        start_write(0, bofs(0), C)

        def pair(k, _):
            i0 = 2 * k
            wait_write(0, bofs(i0 - 2), C)
            start_chunk(0, i0 * C, C)
            wait_chunk(1, C)
            start_write(1, bofs(i0 - 1), C)

            wait_write(1, bofs(i0 - 1), C)
            start_chunk(1, (i0 + 1) * C, C)
            wait_chunk(0, C)
            start_write(0, bofs(i0), C)
            return 0

        lax.fori_loop(1, m // 2, pair, 0)

        wait_chunk(1, C)
        start_write(1, bofs(m - 1), C)
        if tail:
            wait_write(0, bofs(m - 2), C)
            start_chunk(0, m * C, tail)
            wait_chunk(0, tail)
            start_write(0, bofs(m), tail)
            wait_write(0, bofs(m), tail)
        else:
            wait_write(0, bofs(m - 2), C)
        wait_write(1, bofs(m - 1), C)

    return run_stream


def _preload_idx(idx_refs, per_ws, offs, idx_v, si, w):
    for idx_hbm, per_w, off in zip(idx_refs, per_ws, offs):
        pltpu.async_copy(idx_hbm.at[pl.ds(w * per_w, per_w)],
                         idx_v.at[pl.ds(off, per_w)], si)
    for idx_hbm, per_w, off in zip(idx_refs, per_ws, offs):
        pltpu.make_async_copy(idx_hbm.at[pl.ds(0, per_w)],
                              idx_v.at[pl.ds(off, per_w)], si).wait()


def _interleave(out, parity):
    return lambda base, n: out.at[pl.ds(base, n), parity]


def _linear(out):
    return lambda base, n: out.at[pl.ds(base, n)]


PW_A = (N_STATIC // NW, N_STATIC // NW, N_FUT // NW)
OFF_A = (0, PW_A[0], PW_A[0] + PW_A[1])
IDX_A = sum(PW_A)


def _body_a(sc_idx, sca_idx, fu_idx, w_sc, w_sca, w_fu,
            out_s, out_f,
            idx_v, rows_v, sg0, sg1, sw0, sw1, si):
    w = lax.axis_index("s") * NC + lax.axis_index("c")
    _preload_idx((sc_idx, sca_idx, fu_idx), PW_A, OFF_A, idx_v, si, w)
    run = _make_pipeline(idx_v, rows_v, (sg0, sg1), (sw0, sw1), w)
    run(w_sc, OFF_A[0], w * PW_A[0], PW_A[0], _interleave(out_s, 0))
    run(w_sca, OFF_A[1], w * PW_A[1], PW_A[1], _interleave(out_s, 1))
    run(w_fu, OFF_A[2], w * PW_A[2], PW_A[2], _linear(out_f))


PW_B = (N_HIST // NW, N_HIST // NW)
OFF_B = (0, PW_B[0])
IDX_B = sum(PW_B)


def _body_b(hc_idx, hca_idx, w_hc, w_hca,
            out_h,
            idx_v, rows_v, sg0, sg1, sw0, sw1, si):
    w = lax.axis_index("s") * NC + lax.axis_index("c")
    _preload_idx((hc_idx, hca_idx), PW_B, OFF_B, idx_v, si, w)
    run = _make_pipeline(idx_v, rows_v, (sg0, sg1), (sw0, sw1), w)
    run(w_hc, OFF_B[0], w * PW_B[0], PW_B[0], _interleave(out_h, 0))
    run(w_hca, OFF_B[1], w * PW_B[1], PW_B[1], _interleave(out_h, 1))


def _sc_call(body, out_type, idx_words):
    return pl.kernel(
        body,
        out_type=out_type,
        mesh=plsc.VectorSubcoreMesh(**_MESH),
        compiler_params=pltpu.CompilerParams(use_tc_tiling_on_sc=False),
        scratch_types=[
            pltpu.VMEM((idx_words,), jnp.int32),
            pltpu.VMEM((2, C, H), jnp.float32),
            pltpu.SemaphoreType.DMA,
            pltpu.SemaphoreType.DMA,
            pltpu.SemaphoreType.DMA,
            pltpu.SemaphoreType.DMA,
            pltpu.SemaphoreType.DMA,
        ],
    )


@jax.jit
def _embed(sc_idx, sca_idx, hc_idx, hca_idx, fu_idx,
           w_sc, w_sca, w_hc, w_hca, w_fu):
    out_s, out_f = _sc_call(_body_a, [
        jax.ShapeDtypeStruct((N_STATIC, 2, H), jnp.float32),
        jax.ShapeDtypeStruct((N_FUT, H), jnp.float32),
    ], IDX_A)(sc_idx, sca_idx, fu_idx, w_sc, w_sca, w_fu)
    out_h = _sc_call(_body_b, [
        jax.ShapeDtypeStruct((N_HIST, 2, H), jnp.float32),
    ], IDX_B)(hc_idx, hca_idx, w_hc, w_hca)[0]
    return out_s, out_h, out_f


def kernel(static_cont_input, static_cat_input, history_cont_input,
           history_cat_input, future_input, W_static_cont, W_static_cat,
           W_history_cont, W_history_cat, W_future):
    def prep(idx):
        return idx.astype(jnp.int32).reshape(-1)

    out_s, out_h, out_f = _embed(
        prep(static_cont_input), prep(static_cat_input),
        prep(history_cont_input), prep(history_cat_input),
        prep(future_input),
        W_static_cont, W_static_cat, W_history_cont, W_history_cat, W_future)
    return (out_s.reshape(B, 8, 2 * H),
            out_h.reshape(B, 200, 2 * H),
            out_f.reshape(B, 50, H))
